# baseline (speedup 1.0000x reference)
import functools

import jax
import jax.numpy as jnp
from jax import lax
from jax.experimental import pallas as pl
from jax.experimental.pallas import tpu as pltpu
from jax.experimental.pallas import tpu_sc as plsc

_VOCAB = 1000000
_EMB_DIM = 64
_TOTAL = 4096 * 200

_INFO = plsc.get_sparse_core_info()
_NC = _INFO.num_cores
_NS = _INFO.num_subcores
_NW = _NC * _NS

_B_PER_W = _TOTAL // _NW
_CHUNK = 800
_NCHUNK = _B_PER_W // _CHUNK


def _gather_body(idx_hbm, table_hbm, out_hbm, idx_v, rows_v, sem):
    wid = lax.axis_index("s") * _NC + lax.axis_index("c")
    base = wid * _B_PER_W

    @pl.loop(0, _NCHUNK)
    def _chunk(g):
        off = base + g * _CHUNK
        pltpu.sync_copy(idx_hbm.at[pl.ds(off, _CHUNK)], idx_v)
        pltpu.async_copy(table_hbm.at[idx_v], rows_v, sem).wait()
        pltpu.sync_copy(rows_v, out_hbm.at[pl.ds(off, _CHUNK)])


_gather = functools.partial(
    pl.kernel,
    mesh=plsc.VectorSubcoreMesh(core_axis_name="c", subcore_axis_name="s"),
    out_type=jax.ShapeDtypeStruct((_TOTAL, _EMB_DIM), jnp.float32),
    scratch_types=[
        pltpu.VMEM((_CHUNK,), jnp.int32),
        pltpu.VMEM((_CHUNK, _EMB_DIM), jnp.float32),
        pltpu.SemaphoreType.DMA,
    ],
    compiler_params=pltpu.CompilerParams(use_tc_tiling_on_sc=False),
)(_gather_body)


@jax.jit
def kernel(indices, table):
    batch, hist = indices.shape
    flat_idx = indices.reshape(-1).astype(jnp.int32)
    out = _gather(flat_idx, table)
    return out.reshape(batch, hist, table.shape[1])


# speedup vs baseline: 1.0242x; 1.0242x over previous
import functools

import jax
import jax.numpy as jnp
from jax import lax
from jax.experimental import pallas as pl
from jax.experimental.pallas import tpu as pltpu
from jax.experimental.pallas import tpu_sc as plsc

_VOCAB = 1000000
_EMB_DIM = 64
_TOTAL = 4096 * 200

_INFO = plsc.get_sparse_core_info()
_NC = _INFO.num_cores
_NS = _INFO.num_subcores
_NW = _NC * _NS

_B_PER_W = _TOTAL // _NW
_CHUNK = 400
_NBUF = 4
_NCHUNK = _B_PER_W // _CHUNK
assert _NCHUNK % _NBUF == 0


def _gather_body(idx_hbm, table_hbm, out_hbm, idx_all, rows_v, gsem, osem):
    wid = lax.axis_index("s") * _NC + lax.axis_index("c")
    base = wid * _B_PER_W

    def idx_slice(g):
        return idx_all.at[pl.ds(g * _CHUNK, _CHUNK)]

    def gather_start(g, b):
        pltpu.async_copy(table_hbm.at[idx_slice(g)], rows_v.at[b], gsem.at[b])

    def gather_wait(g, b):
        pltpu.make_async_copy(
            table_hbm.at[idx_slice(g)], rows_v.at[b], gsem.at[b]
        ).wait()

    def out_start(g, b):
        pltpu.async_copy(
            rows_v.at[b], out_hbm.at[pl.ds(base + g * _CHUNK, _CHUNK)], osem.at[b]
        )

    def out_wait(g, b):
        pltpu.make_async_copy(
            rows_v.at[b], out_hbm.at[pl.ds(base + g * _CHUNK, _CHUNK)], osem.at[b]
        ).wait()

    pltpu.sync_copy(idx_hbm.at[pl.ds(base, _B_PER_W)], idx_all)

    for b in range(_NBUF):
        gather_start(b, b)

    @pl.loop(0, _NCHUNK - _NBUF, step=_NBUF)
    def _group(g0):
        for b in range(_NBUF):
            g = g0 + b
            gather_wait(g, b)
            out_start(g, b)
            out_wait(g, b)
            gather_start(g + _NBUF, b)

    for b in range(_NBUF):
        g = _NCHUNK - _NBUF + b
        gather_wait(g, b)
        out_start(g, b)
        out_wait(g, b)


_gather = functools.partial(
    pl.kernel,
    mesh=plsc.VectorSubcoreMesh(core_axis_name="c", subcore_axis_name="s"),
    out_type=jax.ShapeDtypeStruct((_TOTAL, _EMB_DIM), jnp.float32),
    scratch_types=[
        pltpu.VMEM((_B_PER_W,), jnp.int32),
        pltpu.VMEM((_NBUF, _CHUNK, _EMB_DIM), jnp.float32),
        pltpu.SemaphoreType.DMA((_NBUF,)),
        pltpu.SemaphoreType.DMA((_NBUF,)),
    ],
    compiler_params=pltpu.CompilerParams(use_tc_tiling_on_sc=False),
)(_gather_body)


@jax.jit
def kernel(indices, table):
    batch, hist = indices.shape
    flat_idx = indices.reshape(-1).astype(jnp.int32)
    out = _gather(flat_idx, table)
    return out.reshape(batch, hist, table.shape[1])
